# Initial kernel scaffold; baseline (speedup 1.0000x reference)
import jax
import jax.numpy as jnp
from jax.experimental import pallas as pl


def kernel(x, edge_index, W1, b1, W3, b3):
    raise NotImplementedError("write your pallas kernel here")


# speedup vs baseline: 8.6354x; 8.6354x over previous
import functools

import jax
import jax.numpy as jnp
from jax import lax
from jax.experimental import pallas as pl
from jax.experimental.pallas import tpu as pltpu
from jax.experimental.pallas import tpu_sc as plsc

N = 10000
NP = 10240
F = 128
H = 256
E = 320000
NC = 2
NS = 16
CK = 128
CH = 160
CG = 8
NG = CH // CG
EP = NS * CH * CK
DUMP = N
BLK = 1024
GRID = NP // BLK
HC = CH // NC
RPS = NP // NS

_vec_mesh = plsc.VectorSubcoreMesh(core_axis_name="c", subcore_axis_name="s")


@functools.partial(
    pl.kernel,
    out_type=jax.ShapeDtypeStruct((NC * NS, NP), jnp.float32),
    mesh=_vec_mesh,
    scratch_types=[
        pltpu.VMEM((HC, CK), jnp.int32),
        pltpu.VMEM((NP,), jnp.float32),
    ],
    compiler_params=pltpu.CompilerParams(needs_layout_passes=False),
)
def _deg_kernel(dst_hbm, out_hbm, dst_buf, deg):
    c = lax.axis_index("c")
    s = lax.axis_index("s")
    pltpu.sync_copy(dst_hbm.at[c * NS + s], dst_buf)
    zeros = jnp.zeros((16,), jnp.float32)

    @pl.loop(0, NP // 16)
    def _(i):
        deg[pl.ds(i * 16, 16)] = zeros

    ones = jnp.ones((16,), jnp.float32)

    @pl.loop(0, HC)
    def _(r):
        for k in range(CK // 16):
            v = dst_buf[r, pl.ds(k * 16, 16)]
            plsc.addupdate_scatter(deg, [v], ones)

    pltpu.sync_copy(deg, out_hbm.at[c * NS + s])


def _make_agg(dh):
    @functools.partial(
        pl.kernel,
        out_type=jax.ShapeDtypeStruct((NC * NP, dh), jnp.float32),
        mesh=_vec_mesh,
        scratch_types=[
            pltpu.VMEM_SHARED((NP, dh), jnp.float32),
            pltpu.VMEM((CG, CK), jnp.int32),
            pltpu.VMEM((CG, CK), jnp.int32),
            pltpu.VMEM((CK, dh), jnp.float32),
            pltpu.SemaphoreType.DMA,
        ],
        compiler_params=pltpu.CompilerParams(
            needs_layout_passes=False, use_tc_tiling_on_sc=False
        ),
    )
    def agg(g_hbm, src_hbm, dst_hbm, out_hbm, accum, src_buf, dst_buf, rows, sem):
        c = lax.axis_index("c")
        s = lax.axis_index("s")
        base = c * NP + s * RPS
        pltpu.sync_copy(g_hbm.at[pl.ds(base, RPS)], accum.at[pl.ds(s * RPS, RPS)])
        plsc.subcore_barrier()

        @pl.loop(0, NG)
        def _(g):
            pltpu.sync_copy(src_hbm.at[c, s, pl.ds(g * CG, CG)], src_buf)
            pltpu.sync_copy(dst_hbm.at[s, pl.ds(g * CG, CG)], dst_buf)

            @pl.loop(0, CG)
            def _(j):
                pltpu.async_copy(g_hbm.at[src_buf.at[j]], rows, sem).wait()
                pltpu.sync_copy(rows, accum.at[dst_buf.at[j]], add=True)

        plsc.subcore_barrier()
        pltpu.sync_copy(accum.at[pl.ds(s * RPS, RPS)], out_hbm.at[pl.ds(base, RPS)])

    return agg


_agg_f = _make_agg(F)
_agg_h = _make_agg(F // 2)


def _dinv_block(degp_blk):
    return lax.rsqrt(jnp.sum(degp_blk, axis=0) + 1.0)


def _b1_body(x_ref, w1_ref, degp_ref, g1_ref):
    dinv = _dinv_block(degp_ref[...])
    h = jnp.dot(x_ref[...], w1_ref[...], preferred_element_type=jnp.float32)
    hs = h * dinv[:, None]
    g1_ref[0] = hs[:, :F]
    g1_ref[1] = hs[:, F:]


def _b2_body(a_ref, w3_ref, b1_ref, degp_ref, g2_ref):
    dinv = _dinv_block(degp_ref[...])
    acat = jnp.concatenate([a_ref[0], a_ref[1]], axis=1)
    out1 = acat * dinv[:, None] + b1_ref[...][None, :]
    h2 = jnp.dot(out1, w3_ref[...], preferred_element_type=jnp.float32)
    hs = h2 * dinv[:, None]
    g2_ref[0] = hs[:, : F // 2]
    g2_ref[1] = hs[:, F // 2 :]


def _b3_body(a_ref, b3_ref, degp_ref, out_ref):
    dinv = _dinv_block(degp_ref[...])
    acat = jnp.concatenate([a_ref[0], a_ref[1]], axis=1)
    out_ref[...] = jax.nn.sigmoid(acat * dinv[:, None] + b3_ref[...][None, :])


_b1 = pl.pallas_call(
    _b1_body,
    grid=(GRID,),
    in_specs=[
        pl.BlockSpec((BLK, F), lambda i: (i, 0)),
        pl.BlockSpec((F, H), lambda i: (0, 0)),
        pl.BlockSpec((NC * NS, BLK), lambda i: (0, i)),
    ],
    out_specs=pl.BlockSpec((2, BLK, F), lambda i: (0, i, 0)),
    out_shape=jax.ShapeDtypeStruct((2, NP, F), jnp.float32),
)

_b2 = pl.pallas_call(
    _b2_body,
    grid=(GRID,),
    in_specs=[
        pl.BlockSpec((2, BLK, F), lambda i: (0, i, 0)),
        pl.BlockSpec((H, F), lambda i: (0, 0)),
        pl.BlockSpec((H,), lambda i: (0,)),
        pl.BlockSpec((NC * NS, BLK), lambda i: (0, i)),
    ],
    out_specs=pl.BlockSpec((2, BLK, F // 2), lambda i: (0, i, 0)),
    out_shape=jax.ShapeDtypeStruct((2, NP, F // 2), jnp.float32),
)

_b3 = pl.pallas_call(
    _b3_body,
    grid=(GRID,),
    in_specs=[
        pl.BlockSpec((2, BLK, F // 2), lambda i: (0, i, 0)),
        pl.BlockSpec((F,), lambda i: (0,)),
        pl.BlockSpec((NC * NS, BLK), lambda i: (0, i)),
    ],
    out_specs=pl.BlockSpec((BLK, F), lambda i: (i, 0)),
    out_shape=jax.ShapeDtypeStruct((NP, F), jnp.float32),
)


def kernel(x, edge_index, W1, b1, W3, b3):
    ei = edge_index.astype(jnp.int32)
    src = jnp.concatenate([ei[0], jnp.zeros((EP - E,), jnp.int32)])
    dst = jnp.concatenate([ei[1], jnp.full((EP - E,), DUMP, jnp.int32)])
    src3 = src.reshape(NS, CH, CK)
    src4 = jnp.stack([src3, src3 + NP])
    dst3 = dst.reshape(NS, CH, CK)
    dst_deg = dst.reshape(NC * NS, HC, CK)
    xp = jnp.pad(x, ((0, NP - N), (0, 0)))

    degp = _deg_kernel(dst_deg)
    g1 = _b1(xp, W1, degp)
    a1 = _agg_f(g1.reshape(NC * NP, F), src4, dst3)
    g2 = _b2(a1.reshape(2, NP, F), W3, b1, degp)
    a2 = _agg_h(g2.reshape(NC * NP, F // 2), src4, dst3)
    out = _b3(a2.reshape(2, NP, F // 2), b3, degp)
    return out[:N]
